# baseline (speedup 1.0000x reference)
import functools

import jax
import jax.numpy as jnp
from jax import lax
from jax.experimental import pallas as pl
from jax.experimental.pallas import tpu as pltpu
from jax.experimental.pallas import tpu_sc as plsc

B = 16384
HID = 128
MEM = 1000000
R = 128
NB = 16
BLK = B // NB

_EPS = 1e-12


def _prefix_rows(p):
    for k in (1, 2, 4, 8, 16, 32, 64):
        p = p + jnp.concatenate([jnp.zeros((R, k), jnp.int32), p[:, : R - k]], axis=1)
    return p


def _prefix_col(p):
    for k in (1, 2, 4, 8, 16, 32, 64):
        p = p + jnp.concatenate([jnp.zeros((k, 1), jnp.int32), p[: R - k, :]], axis=0)
    return p


def _tc_body(x1_ref, x2_ref, va_ref, vb_ref, c_ref, label_ref, score_ref, dest_ref):
    x1 = x1_ref[...]
    x2 = x2_ref[...]
    s1 = jnp.sum(x1 * x1, axis=1, keepdims=True)
    s2 = jnp.sum(x2 * x2, axis=1, keepdims=True)
    d1 = jnp.sum(x1 * va_ref[...], axis=1, keepdims=True)
    d2 = jnp.sum(x2 * vb_ref[...], axis=1, keepdims=True)
    n1 = jnp.maximum(jnp.sqrt(s1), _EPS)
    n2 = jnp.maximum(jnp.sqrt(s2), _EPS)
    score_ref[...] = d1 / n1 + d2 / n2 + c_ref[0, 0]

    @pl.when(pl.program_id(0) == 0)
    def _():
        m = (label_ref[...] <= 0).astype(jnp.int32)
        pm = _prefix_rows(m)
        rs = pm[:, R - 1 : R]
        ic = _prefix_col(rs)
        off = ic - rs
        total = ic[R - 1 : R, 0:1]
        mu = 1 - m
        pu = _prefix_rows(mu)
        rsu = pu[:, R - 1 : R]
        icu = _prefix_col(rsu)
        offu = icu - rsu
        dest_ref[...] = jnp.where(
            m == 1, off + pm - 1, total + offu + pu - 1
        )


_tc_call = pl.pallas_call(
    _tc_body,
    grid=(NB,),
    in_specs=[
        pl.BlockSpec((BLK, HID), lambda i: (i, 0)),
        pl.BlockSpec((BLK, HID), lambda i: (i, 0)),
        pl.BlockSpec((1, HID), lambda i: (0, 0)),
        pl.BlockSpec((1, HID), lambda i: (0, 0)),
        pl.BlockSpec((1, 1), lambda i: (0, 0)),
        pl.BlockSpec((R, R), lambda i: (0, 0)),
    ],
    out_specs=[
        pl.BlockSpec((BLK, 1), lambda i: (i, 0)),
        pl.BlockSpec((R, R), lambda i: (0, 0)),
    ],
    out_shape=[
        jax.ShapeDtypeStruct((B, 1), jnp.float32),
        jax.ShapeDtypeStruct((R, R), jnp.int32),
    ],
)

_NC = 2
_NS = 16
_NW = _NC * _NS
CHUNK = B // _NW


@functools.lru_cache(maxsize=None)
def _sc_scatter_fn():
    mesh = plsc.VectorSubcoreMesh(core_axis_name="c", subcore_axis_name="s")

    @functools.partial(
        pl.kernel,
        mesh=mesh,
        scratch_types=[
            pltpu.VMEM((CHUNK,), jnp.int32),
            pltpu.VMEM((CHUNK,), jnp.float32),
            pltpu.VMEM((CHUNK,), jnp.float32),
            pltpu.VMEM((CHUNK,), jnp.int32),
            pltpu.VMEM((CHUNK,), jnp.float32),
            pltpu.VMEM((CHUNK,), jnp.float32),
            pltpu.VMEM((CHUNK,), jnp.float32),
            pltpu.VMEM((CHUNK,), jnp.float32),
            pltpu.SemaphoreType.DMA,
        ],
    )
    def _sc_scatter(dest_hbm, score_hbm, time_hbm, label_hbm, mem_ref, tmem_ref,
                    dest_v, score_v, time_v, label_v, oldm_v, oldt_v,
                    valm_v, valt_v, sem):
        wid = lax.axis_index("s") * _NC + lax.axis_index("c")
        base = wid * CHUNK
        pltpu.sync_copy(dest_hbm.at[pl.ds(base, CHUNK)], dest_v)
        pltpu.sync_copy(score_hbm.at[pl.ds(base, CHUNK)], score_v)
        pltpu.sync_copy(time_hbm.at[pl.ds(base, CHUNK)], time_v)
        pltpu.sync_copy(label_hbm.at[pl.ds(base, CHUNK)], label_v)
        g1 = pltpu.async_copy(mem_ref.at[dest_v], oldm_v, sem)
        g2 = pltpu.async_copy(tmem_ref.at[dest_v], oldt_v, sem)
        g1.wait()
        g2.wait()
        for i in range(CHUNK // 16):
            sl = pl.ds(i * 16, 16)
            msk = label_v[sl] <= 0
            valm_v[sl] = jnp.where(msk, score_v[sl], oldm_v[sl])
            valt_v[sl] = jnp.where(msk, time_v[sl], oldt_v[sl])
        s1 = pltpu.async_copy(valm_v, mem_ref.at[dest_v], sem)
        s2 = pltpu.async_copy(valt_v, tmem_ref.at[dest_v], sem)
        s1.wait()
        s2.wait()

    return _sc_scatter


def kernel(x1, x2, time, label, w1, b1, w2, b2, memory, time_memory):
    v = w2 @ w1
    va = v[:, :HID]
    vb = v[:, HID:]
    carr = (jnp.dot(b1, w2[0]) + b2[0]).reshape(1, 1)
    label2d = label.astype(jnp.int32).reshape(R, R)
    score, dest2d = _tc_call(x1, x2, va, vb, carr, label2d)
    mem_ref = jax.new_ref(memory)
    tmem_ref = jax.new_ref(time_memory)
    _sc_scatter_fn()(
        dest2d.reshape(B), score.reshape(B), time.reshape(B),
        label.astype(jnp.int32), mem_ref, tmem_ref)
    return score, mem_ref[...], tmem_ref[...]


# speedup vs baseline: 2.2386x; 2.2386x over previous
import functools

import jax
import jax.numpy as jnp
from jax import lax
from jax.experimental import pallas as pl
from jax.experimental.pallas import tpu as pltpu
from jax.experimental.pallas import tpu_sc as plsc

B = 16384
HID = 128
MEM = 1000000
R = 128
NB = 16
BLK = B // NB

_EPS = 1e-12


def _prefix_rows(p):
    for k in (1, 2, 4, 8, 16, 32, 64):
        p = p + jnp.concatenate([jnp.zeros((R, k), jnp.int32), p[:, : R - k]], axis=1)
    return p


def _prefix_col(p):
    for k in (1, 2, 4, 8, 16, 32, 64):
        p = p + jnp.concatenate([jnp.zeros((k, 1), jnp.int32), p[: R - k, :]], axis=0)
    return p


def _tc_body(x1_ref, x2_ref, va_ref, vb_ref, c_ref, label_ref, score_ref, dest_ref):
    x1 = x1_ref[...]
    x2 = x2_ref[...]
    s1 = jnp.sum(x1 * x1, axis=1, keepdims=True)
    s2 = jnp.sum(x2 * x2, axis=1, keepdims=True)
    d1 = jnp.sum(x1 * va_ref[...], axis=1, keepdims=True)
    d2 = jnp.sum(x2 * vb_ref[...], axis=1, keepdims=True)
    n1 = jnp.maximum(jnp.sqrt(s1), _EPS)
    n2 = jnp.maximum(jnp.sqrt(s2), _EPS)
    score_ref[...] = d1 / n1 + d2 / n2 + c_ref[0, 0]

    @pl.when(pl.program_id(0) == 0)
    def _():
        m = (label_ref[...] <= 0).astype(jnp.int32)
        pm = _prefix_rows(m)
        rs = pm[:, R - 1 : R]
        ic = _prefix_col(rs)
        off = ic - rs
        total = ic[R - 1 : R, 0:1]
        mu = 1 - m
        pu = _prefix_rows(mu)
        rsu = pu[:, R - 1 : R]
        icu = _prefix_col(rsu)
        offu = icu - rsu
        dest_ref[...] = jnp.where(
            m == 1, off + pm - 1, total + offu + pu - 1
        )


_tc_call = pl.pallas_call(
    _tc_body,
    grid=(NB,),
    in_specs=[
        pl.BlockSpec((BLK, HID), lambda i: (i, 0)),
        pl.BlockSpec((BLK, HID), lambda i: (i, 0)),
        pl.BlockSpec((1, HID), lambda i: (0, 0)),
        pl.BlockSpec((1, HID), lambda i: (0, 0)),
        pl.BlockSpec((1, 1), lambda i: (0, 0)),
        pl.BlockSpec((R, R), lambda i: (0, 0)),
    ],
    out_specs=[
        pl.BlockSpec((BLK, 1), lambda i: (i, 0)),
        pl.BlockSpec((R, R), lambda i: (0, 0)),
    ],
    out_shape=[
        jax.ShapeDtypeStruct((B, 1), jnp.float32),
        jax.ShapeDtypeStruct((R, R), jnp.int32),
    ],
)

_NC = 2
_NS = 16
_NW = _NC * _NS
CHUNK = B // _NS
HALF = B // _NC
STAGE = HALF // _NS


@functools.lru_cache(maxsize=None)
def _sc_scatter_fn():
    mesh = plsc.VectorSubcoreMesh(core_axis_name="c", subcore_axis_name="s")

    @functools.partial(
        pl.kernel,
        mesh=mesh,
        scratch_types=[
            pltpu.VMEM((CHUNK,), jnp.int32),
            pltpu.VMEM((CHUNK,), jnp.float32),
            pltpu.VMEM((CHUNK,), jnp.float32),
            pltpu.VMEM((CHUNK,), jnp.int32),
            pltpu.VMEM((CHUNK,), jnp.int32),
            pltpu.VMEM_SHARED((HALF + 64,), jnp.float32),
            pltpu.VMEM_SHARED((HALF + 64,), jnp.float32),
            pltpu.SemaphoreType.DMA,
        ],
    )
    def _sc_scatter(dest_hbm, score_hbm, time_hbm, label_hbm, mem_ref, tmem_ref,
                    dest_v, score_v, time_v, label_v, idx_v,
                    bufm_s, buft_s, sem):
        c = lax.axis_index("c")
        s = lax.axis_index("s")
        rbase = s * CHUNK
        pltpu.sync_copy(dest_hbm.at[pl.ds(rbase, CHUNK)], dest_v)
        pltpu.sync_copy(score_hbm.at[pl.ds(rbase, CHUNK)], score_v)
        pltpu.sync_copy(time_hbm.at[pl.ds(rbase, CHUNK)], time_v)
        pltpu.sync_copy(label_hbm.at[pl.ds(rbase, CHUNK)], label_v)
        hstage = c * HALF + s * STAGE
        pltpu.sync_copy(mem_ref.at[pl.ds(hstage, STAGE)],
                        bufm_s.at[pl.ds(s * STAGE, STAGE)])
        pltpu.sync_copy(tmem_ref.at[pl.ds(hstage, STAGE)],
                        buft_s.at[pl.ds(s * STAGE, STAGE)])
        base = c * HALF
        for i in range(CHUNK // 16):
            sl = pl.ds(i * 16, 16)
            d = dest_v[sl]
            keep = (label_v[sl] <= 0) & (d >= base) & (d < base + HALF)
            idx_v[sl] = jnp.where(keep, d - base, HALF)
        plsc.subcore_barrier()
        s1 = pltpu.async_copy(score_v, bufm_s.at[idx_v], sem)
        s2 = pltpu.async_copy(time_v, buft_s.at[idx_v], sem)
        s1.wait()
        s2.wait()
        plsc.subcore_barrier()
        pltpu.sync_copy(bufm_s.at[pl.ds(s * STAGE, STAGE)],
                        mem_ref.at[pl.ds(hstage, STAGE)])
        pltpu.sync_copy(buft_s.at[pl.ds(s * STAGE, STAGE)],
                        tmem_ref.at[pl.ds(hstage, STAGE)])

    return _sc_scatter


def kernel(x1, x2, time, label, w1, b1, w2, b2, memory, time_memory):
    v = w2 @ w1
    va = v[:, :HID]
    vb = v[:, HID:]
    carr = (jnp.dot(b1, w2[0]) + b2[0]).reshape(1, 1)
    label2d = label.astype(jnp.int32).reshape(R, R)
    score, dest2d = _tc_call(x1, x2, va, vb, carr, label2d)
    mem_ref = jax.new_ref(memory)
    tmem_ref = jax.new_ref(time_memory)
    _sc_scatter_fn()(
        dest2d.reshape(B), score.reshape(B), time.reshape(B),
        label.astype(jnp.int32), mem_ref, tmem_ref)
    return score, mem_ref[...], tmem_ref[...]


# speedup vs baseline: 2.5913x; 1.1576x over previous
import functools

import jax
import jax.numpy as jnp
from jax import lax
from jax.experimental import pallas as pl
from jax.experimental.pallas import tpu as pltpu
from jax.experimental.pallas import tpu_sc as plsc

B = 16384
HID = 128
MEM = 1000000
R = 128
NB = 16
BLK = B // NB

_EPS = 1e-12


def _prefix_rows(p):
    for k in (1, 2, 4, 8, 16, 32, 64):
        p = p + jnp.concatenate([jnp.zeros((R, k), jnp.int32), p[:, : R - k]], axis=1)
    return p


def _prefix_col(p):
    for k in (1, 2, 4, 8, 16, 32, 64):
        p = p + jnp.concatenate([jnp.zeros((k, 1), jnp.int32), p[: R - k, :]], axis=0)
    return p


def _tc_body(x1_ref, x2_ref, va_ref, vb_ref, c_ref, label_ref, score_ref, dest_ref):
    x1 = x1_ref[...]
    x2 = x2_ref[...]
    ones_row = jnp.ones((1, HID), jnp.float32)
    dn = (((1,), (1,)), ((), ()))
    d1 = lax.dot_general(va_ref[...], x1, dn, preferred_element_type=jnp.float32)
    s1 = lax.dot_general(ones_row, x1 * x1, dn, preferred_element_type=jnp.float32)
    d2 = lax.dot_general(vb_ref[...], x2, dn, preferred_element_type=jnp.float32)
    s2 = lax.dot_general(ones_row, x2 * x2, dn, preferred_element_type=jnp.float32)
    n1 = jnp.maximum(jnp.sqrt(s1), _EPS)
    n2 = jnp.maximum(jnp.sqrt(s2), _EPS)
    score_ref[...] = d1 / n1 + d2 / n2 + c_ref[0, 0]

    @pl.when(pl.program_id(0) == 0)
    def _():
        m = (label_ref[...] <= 0).astype(jnp.int32)
        pm = _prefix_rows(m)
        rs = pm[:, R - 1 : R]
        ic = _prefix_col(rs)
        off = ic - rs
        total = ic[R - 1 : R, 0:1]
        mu = 1 - m
        pu = _prefix_rows(mu)
        rsu = pu[:, R - 1 : R]
        icu = _prefix_col(rsu)
        offu = icu - rsu
        dest_ref[...] = jnp.where(
            m == 1, off + pm - 1, total + offu + pu - 1
        )


_tc_call = pl.pallas_call(
    _tc_body,
    grid=(NB,),
    in_specs=[
        pl.BlockSpec((BLK, HID), lambda i: (i, 0)),
        pl.BlockSpec((BLK, HID), lambda i: (i, 0)),
        pl.BlockSpec((1, HID), lambda i: (0, 0)),
        pl.BlockSpec((1, HID), lambda i: (0, 0)),
        pl.BlockSpec((1, 1), lambda i: (0, 0)),
        pl.BlockSpec((R, R), lambda i: (0, 0)),
    ],
    out_specs=[
        pl.BlockSpec((1, BLK), lambda i: (0, i)),
        pl.BlockSpec((R, R), lambda i: (0, 0)),
    ],
    out_shape=[
        jax.ShapeDtypeStruct((1, B), jnp.float32),
        jax.ShapeDtypeStruct((R, R), jnp.int32),
    ],
)

_NC = 2
_NS = 16
_NW = _NC * _NS
CHUNK = B // _NS
HALF = B // _NC
STAGE = HALF // _NS


@functools.lru_cache(maxsize=None)
def _sc_scatter_fn():
    mesh = plsc.VectorSubcoreMesh(core_axis_name="c", subcore_axis_name="s")

    @functools.partial(
        pl.kernel,
        mesh=mesh,
        scratch_types=[
            pltpu.VMEM((CHUNK,), jnp.int32),
            pltpu.VMEM((CHUNK,), jnp.float32),
            pltpu.VMEM((CHUNK,), jnp.float32),
            pltpu.VMEM((CHUNK,), jnp.int32),
            pltpu.VMEM((CHUNK,), jnp.int32),
            pltpu.VMEM_SHARED((HALF + 64,), jnp.float32),
            pltpu.VMEM_SHARED((HALF + 64,), jnp.float32),
            pltpu.SemaphoreType.DMA,
        ],
    )
    def _sc_scatter(dest_hbm, score_hbm, time_hbm, label_hbm, mem_ref, tmem_ref,
                    dest_v, score_v, time_v, label_v, idx_v,
                    bufm_s, buft_s, sem):
        c = lax.axis_index("c")
        s = lax.axis_index("s")
        rbase = s * CHUNK
        pltpu.sync_copy(dest_hbm.at[pl.ds(rbase, CHUNK)], dest_v)
        pltpu.sync_copy(score_hbm.at[pl.ds(rbase, CHUNK)], score_v)
        pltpu.sync_copy(time_hbm.at[pl.ds(rbase, CHUNK)], time_v)
        pltpu.sync_copy(label_hbm.at[pl.ds(rbase, CHUNK)], label_v)
        hstage = c * HALF + s * STAGE
        pltpu.sync_copy(mem_ref.at[pl.ds(hstage, STAGE)],
                        bufm_s.at[pl.ds(s * STAGE, STAGE)])
        pltpu.sync_copy(tmem_ref.at[pl.ds(hstage, STAGE)],
                        buft_s.at[pl.ds(s * STAGE, STAGE)])
        base = c * HALF
        for i in range(CHUNK // 16):
            sl = pl.ds(i * 16, 16)
            d = dest_v[sl]
            keep = (label_v[sl] <= 0) & (d >= base) & (d < base + HALF)
            idx_v[sl] = jnp.where(keep, d - base, HALF)
        plsc.subcore_barrier()
        s1 = pltpu.async_copy(score_v, bufm_s.at[idx_v], sem)
        s2 = pltpu.async_copy(time_v, buft_s.at[idx_v], sem)
        s1.wait()
        s2.wait()
        plsc.subcore_barrier()
        pltpu.sync_copy(bufm_s.at[pl.ds(s * STAGE, STAGE)],
                        mem_ref.at[pl.ds(hstage, STAGE)])
        pltpu.sync_copy(buft_s.at[pl.ds(s * STAGE, STAGE)],
                        tmem_ref.at[pl.ds(hstage, STAGE)])

    return _sc_scatter


def kernel(x1, x2, time, label, w1, b1, w2, b2, memory, time_memory):
    v = w2 @ w1
    va = v[:, :HID]
    vb = v[:, HID:]
    carr = (jnp.dot(b1, w2[0]) + b2[0]).reshape(1, 1)
    label2d = label.astype(jnp.int32).reshape(R, R)
    score, dest2d = _tc_call(x1, x2, va, vb, carr, label2d)
    mem_ref = jax.new_ref(memory)
    tmem_ref = jax.new_ref(time_memory)
    _sc_scatter_fn()(
        dest2d.reshape(B), score.reshape(B), time.reshape(B),
        label.astype(jnp.int32), mem_ref, tmem_ref)
    return score.reshape(B, 1), mem_ref[...], tmem_ref[...]


# speedup vs baseline: 2.7187x; 1.0492x over previous
import functools

import jax
import jax.numpy as jnp
from jax import lax
from jax.experimental import pallas as pl
from jax.experimental.pallas import tpu as pltpu
from jax.experimental.pallas import tpu_sc as plsc

B = 16384
HID = 128
MEM = 1000000
R = 128
NB = 16
BLK = B // NB

_EPS = 1e-12


def _prefix_rows(p):
    for k in (1, 2, 4, 8, 16, 32, 64):
        p = p + jnp.concatenate([jnp.zeros((R, k), jnp.int32), p[:, : R - k]], axis=1)
    return p


def _prefix_col(p):
    for k in (1, 2, 4, 8, 16, 32, 64):
        p = p + jnp.concatenate([jnp.zeros((k, 1), jnp.int32), p[: R - k, :]], axis=0)
    return p


def _tc_body(x1_ref, x2_ref, va_ref, vb_ref, c_ref, label_ref, score_ref, dest_ref):
    x1 = x1_ref[...]
    x2 = x2_ref[...]
    ones_row = jnp.ones((1, HID), jnp.float32)
    dn = (((1,), (1,)), ((), ()))
    d1 = lax.dot_general(va_ref[...], x1, dn, preferred_element_type=jnp.float32)
    s1 = lax.dot_general(ones_row, x1 * x1, dn, preferred_element_type=jnp.float32)
    d2 = lax.dot_general(vb_ref[...], x2, dn, preferred_element_type=jnp.float32)
    s2 = lax.dot_general(ones_row, x2 * x2, dn, preferred_element_type=jnp.float32)
    n1 = jnp.maximum(jnp.sqrt(s1), _EPS)
    n2 = jnp.maximum(jnp.sqrt(s2), _EPS)
    score_ref[...] = d1 / n1 + d2 / n2 + c_ref[0, 0]

    @pl.when(pl.program_id(0) == 0)
    def _():
        m = (label_ref[...] <= 0).astype(jnp.int32)
        pm = _prefix_rows(m)
        rs = pm[:, R - 1 : R]
        ic = _prefix_col(rs)
        off = ic - rs
        total = ic[R - 1 : R, 0:1]
        mu = 1 - m
        pu = _prefix_rows(mu)
        rsu = pu[:, R - 1 : R]
        icu = _prefix_col(rsu)
        offu = icu - rsu
        dest_ref[...] = jnp.where(
            m == 1, off + pm - 1, total + offu + pu - 1
        )


_tc_call = pl.pallas_call(
    _tc_body,
    grid=(NB,),
    in_specs=[
        pl.BlockSpec((BLK, HID), lambda i: (i, 0)),
        pl.BlockSpec((BLK, HID), lambda i: (i, 0)),
        pl.BlockSpec((1, HID), lambda i: (0, 0)),
        pl.BlockSpec((1, HID), lambda i: (0, 0)),
        pl.BlockSpec((1, 1), lambda i: (0, 0)),
        pl.BlockSpec((R, R), lambda i: (0, 0)),
    ],
    out_specs=[
        pl.BlockSpec((1, BLK), lambda i: (0, i)),
        pl.BlockSpec((R, R), lambda i: (0, 0)),
    ],
    out_shape=[
        jax.ShapeDtypeStruct((1, B), jnp.float32),
        jax.ShapeDtypeStruct((R, R), jnp.int32),
    ],
)

_NC = 2
_NS = 16
_NW = _NC * _NS
CHUNK = B // _NS
HALF = B // _NC
STAGE = HALF // _NS


@functools.lru_cache(maxsize=None)
def _sc_scatter_fn():
    mesh = plsc.VectorSubcoreMesh(core_axis_name="c", subcore_axis_name="s")

    @functools.partial(
        pl.kernel,
        mesh=mesh,
        scratch_types=[
            pltpu.VMEM((CHUNK,), jnp.int32),
            pltpu.VMEM((CHUNK,), jnp.float32),
            pltpu.VMEM((CHUNK,), jnp.float32),
            pltpu.VMEM((CHUNK,), jnp.int32),
            pltpu.VMEM((CHUNK,), jnp.int32),
            pltpu.VMEM_SHARED((HALF + 64,), jnp.float32),
            pltpu.VMEM_SHARED((HALF + 64,), jnp.float32),
            pltpu.SemaphoreType.DMA,
        ],
    )
    def _sc_scatter(dest_hbm, score_hbm, time_hbm, label_hbm, mem_ref, tmem_ref,
                    dest_v, score_v, time_v, label_v, idx_v,
                    bufm_s, buft_s, sem):
        c = lax.axis_index("c")
        s = lax.axis_index("s")
        rbase = s * CHUNK
        hstage = c * HALF + s * STAGE
        loads = [
            pltpu.async_copy(dest_hbm.at[pl.ds(rbase, CHUNK)], dest_v, sem),
            pltpu.async_copy(score_hbm.at[pl.ds(rbase, CHUNK)], score_v, sem),
            pltpu.async_copy(time_hbm.at[pl.ds(rbase, CHUNK)], time_v, sem),
            pltpu.async_copy(label_hbm.at[pl.ds(rbase, CHUNK)], label_v, sem),
            pltpu.async_copy(mem_ref.at[pl.ds(hstage, STAGE)],
                             bufm_s.at[pl.ds(s * STAGE, STAGE)], sem),
            pltpu.async_copy(tmem_ref.at[pl.ds(hstage, STAGE)],
                             buft_s.at[pl.ds(s * STAGE, STAGE)], sem),
        ]
        for ld in loads:
            ld.wait()
        base = c * HALF
        for i in range(CHUNK // 16):
            sl = pl.ds(i * 16, 16)
            d = dest_v[sl]
            keep = (label_v[sl] <= 0) & (d >= base) & (d < base + HALF)
            idx_v[sl] = jnp.where(keep, d - base, HALF)
        plsc.subcore_barrier()
        s1 = pltpu.async_copy(score_v, bufm_s.at[idx_v], sem)
        s2 = pltpu.async_copy(time_v, buft_s.at[idx_v], sem)
        s1.wait()
        s2.wait()
        plsc.subcore_barrier()
        w1_ = pltpu.async_copy(bufm_s.at[pl.ds(s * STAGE, STAGE)],
                               mem_ref.at[pl.ds(hstage, STAGE)], sem)
        w2_ = pltpu.async_copy(buft_s.at[pl.ds(s * STAGE, STAGE)],
                               tmem_ref.at[pl.ds(hstage, STAGE)], sem)
        w1_.wait()
        w2_.wait()

    return _sc_scatter


def kernel(x1, x2, time, label, w1, b1, w2, b2, memory, time_memory):
    v = w2 @ w1
    va = v[:, :HID]
    vb = v[:, HID:]
    carr = (jnp.dot(b1, w2[0]) + b2[0]).reshape(1, 1)
    label2d = label.astype(jnp.int32).reshape(R, R)
    score, dest2d = _tc_call(x1, x2, va, vb, carr, label2d)
    mem_ref = jax.new_ref(memory)
    tmem_ref = jax.new_ref(time_memory)
    _sc_scatter_fn()(
        dest2d.reshape(B), score.reshape(B), time.reshape(B),
        label.astype(jnp.int32), mem_ref, tmem_ref)
    return score.reshape(B, 1), mem_ref[...], tmem_ref[...]


# speedup vs baseline: 3.0150x; 1.1090x over previous
import functools

import jax
import jax.numpy as jnp
from jax import lax
from jax.experimental import pallas as pl
from jax.experimental.pallas import tpu as pltpu
from jax.experimental.pallas import tpu_sc as plsc

B = 16384
HID = 128
MEM = 1000000
R = 128
NB = 4
BLK = B // NB

_EPS = 1e-12


def _prefix_rows(p):
    for k in (1, 2, 4, 8, 16, 32, 64):
        p = p + jnp.concatenate([jnp.zeros((R, k), jnp.int32), p[:, : R - k]], axis=1)
    return p


def _prefix_col(p):
    for k in (1, 2, 4, 8, 16, 32, 64):
        p = p + jnp.concatenate([jnp.zeros((k, 1), jnp.int32), p[: R - k, :]], axis=0)
    return p


def _tc_body(x1_ref, x2_ref, va_ref, vb_ref, c_ref, label_ref, score_ref, dest_ref):
    x1 = x1_ref[...]
    x2 = x2_ref[...]
    ones_row = jnp.ones((1, HID), jnp.float32)
    dn = (((1,), (1,)), ((), ()))
    d1 = lax.dot_general(va_ref[...], x1, dn, preferred_element_type=jnp.float32)
    s1 = lax.dot_general(ones_row, x1 * x1, dn, preferred_element_type=jnp.float32)
    d2 = lax.dot_general(vb_ref[...], x2, dn, preferred_element_type=jnp.float32)
    s2 = lax.dot_general(ones_row, x2 * x2, dn, preferred_element_type=jnp.float32)
    n1 = jnp.maximum(jnp.sqrt(s1), _EPS)
    n2 = jnp.maximum(jnp.sqrt(s2), _EPS)
    score_ref[...] = d1 / n1 + d2 / n2 + c_ref[0, 0]

    @pl.when(pl.program_id(0) == 0)
    def _():
        m = (label_ref[...] <= 0).astype(jnp.int32)
        pm = _prefix_rows(m)
        rs = pm[:, R - 1 : R]
        ic = _prefix_col(rs)
        off = ic - rs
        total = ic[R - 1 : R, 0:1]
        mu = 1 - m
        pu = _prefix_rows(mu)
        rsu = pu[:, R - 1 : R]
        icu = _prefix_col(rsu)
        offu = icu - rsu
        dest_ref[...] = jnp.where(
            m == 1, off + pm - 1, total + offu + pu - 1
        )


_tc_call = pl.pallas_call(
    _tc_body,
    grid=(NB,),
    in_specs=[
        pl.BlockSpec((BLK, HID), lambda i: (i, 0)),
        pl.BlockSpec((BLK, HID), lambda i: (i, 0)),
        pl.BlockSpec((1, HID), lambda i: (0, 0)),
        pl.BlockSpec((1, HID), lambda i: (0, 0)),
        pl.BlockSpec((1, 1), lambda i: (0, 0)),
        pl.BlockSpec((R, R), lambda i: (0, 0)),
    ],
    out_specs=[
        pl.BlockSpec((1, BLK), lambda i: (0, i)),
        pl.BlockSpec((R, R), lambda i: (0, 0)),
    ],
    out_shape=[
        jax.ShapeDtypeStruct((1, B), jnp.float32),
        jax.ShapeDtypeStruct((R, R), jnp.int32),
    ],
)

_NC = 2
_NS = 16
_NW = _NC * _NS
CHUNK = B // _NS
HALF = B // _NC
STAGE = HALF // _NS


@functools.lru_cache(maxsize=None)
def _sc_scatter_fn():
    mesh = plsc.VectorSubcoreMesh(core_axis_name="c", subcore_axis_name="s")

    @functools.partial(
        pl.kernel,
        mesh=mesh,
        scratch_types=[
            pltpu.VMEM((CHUNK,), jnp.int32),
            pltpu.VMEM((CHUNK,), jnp.float32),
            pltpu.VMEM((CHUNK,), jnp.float32),
            pltpu.VMEM((CHUNK,), jnp.int32),
            pltpu.VMEM((CHUNK,), jnp.int32),
            pltpu.VMEM_SHARED((HALF + 64,), jnp.float32),
            pltpu.VMEM_SHARED((HALF + 64,), jnp.float32),
            pltpu.SemaphoreType.DMA,
        ],
    )
    def _sc_scatter(dest_hbm, score_hbm, time_hbm, label_hbm, mem_ref, tmem_ref,
                    dest_v, score_v, time_v, label_v, idx_v,
                    bufm_s, buft_s, sem):
        c = lax.axis_index("c")
        s = lax.axis_index("s")
        rbase = s * CHUNK
        hstage = c * HALF + s * STAGE
        loads = [
            pltpu.async_copy(dest_hbm.at[pl.ds(rbase, CHUNK)], dest_v, sem),
            pltpu.async_copy(score_hbm.at[pl.ds(rbase, CHUNK)], score_v, sem),
            pltpu.async_copy(time_hbm.at[pl.ds(rbase, CHUNK)], time_v, sem),
            pltpu.async_copy(label_hbm.at[pl.ds(rbase, CHUNK)], label_v, sem),
            pltpu.async_copy(mem_ref.at[pl.ds(hstage, STAGE)],
                             bufm_s.at[pl.ds(s * STAGE, STAGE)], sem),
            pltpu.async_copy(tmem_ref.at[pl.ds(hstage, STAGE)],
                             buft_s.at[pl.ds(s * STAGE, STAGE)], sem),
        ]
        for ld in loads:
            ld.wait()
        base = c * HALF
        for i in range(CHUNK // 16):
            sl = pl.ds(i * 16, 16)
            d = dest_v[sl]
            keep = (label_v[sl] <= 0) & (d >= base) & (d < base + HALF)
            idx_v[sl] = jnp.where(keep, d - base, HALF)
        plsc.subcore_barrier()
        s1 = pltpu.async_copy(score_v, bufm_s.at[idx_v], sem)
        s2 = pltpu.async_copy(time_v, buft_s.at[idx_v], sem)
        s1.wait()
        s2.wait()
        plsc.subcore_barrier()
        w1_ = pltpu.async_copy(bufm_s.at[pl.ds(s * STAGE, STAGE)],
                               mem_ref.at[pl.ds(hstage, STAGE)], sem)
        w2_ = pltpu.async_copy(buft_s.at[pl.ds(s * STAGE, STAGE)],
                               tmem_ref.at[pl.ds(hstage, STAGE)], sem)
        w1_.wait()
        w2_.wait()

    return _sc_scatter


def kernel(x1, x2, time, label, w1, b1, w2, b2, memory, time_memory):
    v = w2 @ w1
    va = v[:, :HID]
    vb = v[:, HID:]
    carr = (jnp.dot(b1, w2[0]) + b2[0]).reshape(1, 1)
    label2d = label.astype(jnp.int32).reshape(R, R)
    score, dest2d = _tc_call(x1, x2, va, vb, carr, label2d)
    mem_ref = jax.new_ref(memory)
    tmem_ref = jax.new_ref(time_memory)
    _sc_scatter_fn()(
        dest2d.reshape(B), score.reshape(B), time.reshape(B),
        label.astype(jnp.int32), mem_ref, tmem_ref)
    return score.reshape(B, 1), mem_ref[...], tmem_ref[...]


# speedup vs baseline: 3.0204x; 1.0018x over previous
import functools

import jax
import jax.numpy as jnp
from jax import lax
from jax.experimental import pallas as pl
from jax.experimental.pallas import tpu as pltpu
from jax.experimental.pallas import tpu_sc as plsc

B = 16384
HID = 128
MEM = 1000000
R = 128
NB = 2
BLK = B // NB

_EPS = 1e-12


def _prefix_rows(p):
    for k in (1, 2, 4, 8, 16, 32, 64):
        p = p + jnp.concatenate([jnp.zeros((R, k), jnp.int32), p[:, : R - k]], axis=1)
    return p


def _prefix_col(p):
    for k in (1, 2, 4, 8, 16, 32, 64):
        p = p + jnp.concatenate([jnp.zeros((k, 1), jnp.int32), p[: R - k, :]], axis=0)
    return p


def _tc_body(x1_ref, x2_ref, va_ref, vb_ref, c_ref, label_ref, score_ref, dest_ref):
    x1 = x1_ref[...]
    x2 = x2_ref[...]
    ones_row = jnp.ones((1, HID), jnp.float32)
    dn = (((1,), (1,)), ((), ()))
    d1 = lax.dot_general(va_ref[...], x1, dn, preferred_element_type=jnp.float32)
    s1 = lax.dot_general(ones_row, x1 * x1, dn, preferred_element_type=jnp.float32)
    d2 = lax.dot_general(vb_ref[...], x2, dn, preferred_element_type=jnp.float32)
    s2 = lax.dot_general(ones_row, x2 * x2, dn, preferred_element_type=jnp.float32)
    n1 = jnp.maximum(jnp.sqrt(s1), _EPS)
    n2 = jnp.maximum(jnp.sqrt(s2), _EPS)
    score_ref[...] = d1 / n1 + d2 / n2 + c_ref[0, 0]

    @pl.when(pl.program_id(0) == 0)
    def _():
        m = (label_ref[...] <= 0).astype(jnp.int32)
        pm = _prefix_rows(m)
        rs = pm[:, R - 1 : R]
        ic = _prefix_col(rs)
        off = ic - rs
        total = ic[R - 1 : R, 0:1]
        mu = 1 - m
        pu = _prefix_rows(mu)
        rsu = pu[:, R - 1 : R]
        icu = _prefix_col(rsu)
        offu = icu - rsu
        dest_ref[...] = jnp.where(
            m == 1, off + pm - 1, total + offu + pu - 1
        )


_tc_call = pl.pallas_call(
    _tc_body,
    grid=(NB,),
    in_specs=[
        pl.BlockSpec((BLK, HID), lambda i: (i, 0)),
        pl.BlockSpec((BLK, HID), lambda i: (i, 0)),
        pl.BlockSpec((1, HID), lambda i: (0, 0)),
        pl.BlockSpec((1, HID), lambda i: (0, 0)),
        pl.BlockSpec((1, 1), lambda i: (0, 0)),
        pl.BlockSpec((R, R), lambda i: (0, 0)),
    ],
    out_specs=[
        pl.BlockSpec((1, BLK), lambda i: (0, i)),
        pl.BlockSpec((R, R), lambda i: (0, 0)),
    ],
    out_shape=[
        jax.ShapeDtypeStruct((1, B), jnp.float32),
        jax.ShapeDtypeStruct((R, R), jnp.int32),
    ],
)

_NC = 2
_NS = 16
_NW = _NC * _NS
CHUNK = B // _NS
HALF = B // _NC
STAGE = HALF // _NS


@functools.lru_cache(maxsize=None)
def _sc_scatter_fn():
    mesh = plsc.VectorSubcoreMesh(core_axis_name="c", subcore_axis_name="s")

    @functools.partial(
        pl.kernel,
        mesh=mesh,
        scratch_types=[
            pltpu.VMEM((CHUNK,), jnp.int32),
            pltpu.VMEM((CHUNK,), jnp.float32),
            pltpu.VMEM((CHUNK,), jnp.float32),
            pltpu.VMEM((CHUNK,), jnp.int32),
            pltpu.VMEM((CHUNK,), jnp.int32),
            pltpu.VMEM_SHARED((HALF + 64,), jnp.float32),
            pltpu.VMEM_SHARED((HALF + 64,), jnp.float32),
            pltpu.SemaphoreType.DMA,
        ],
    )
    def _sc_scatter(dest_hbm, score_hbm, time_hbm, label_hbm, mem_ref, tmem_ref,
                    dest_v, score_v, time_v, label_v, idx_v,
                    bufm_s, buft_s, sem):
        c = lax.axis_index("c")
        s = lax.axis_index("s")
        rbase = s * CHUNK
        hstage = c * HALF + s * STAGE
        loads = [
            pltpu.async_copy(dest_hbm.at[pl.ds(rbase, CHUNK)], dest_v, sem),
            pltpu.async_copy(score_hbm.at[pl.ds(rbase, CHUNK)], score_v, sem),
            pltpu.async_copy(time_hbm.at[pl.ds(rbase, CHUNK)], time_v, sem),
            pltpu.async_copy(label_hbm.at[pl.ds(rbase, CHUNK)], label_v, sem),
            pltpu.async_copy(mem_ref.at[pl.ds(hstage, STAGE)],
                             bufm_s.at[pl.ds(s * STAGE, STAGE)], sem),
            pltpu.async_copy(tmem_ref.at[pl.ds(hstage, STAGE)],
                             buft_s.at[pl.ds(s * STAGE, STAGE)], sem),
        ]
        for ld in loads:
            ld.wait()
        base = c * HALF
        for i in range(CHUNK // 16):
            sl = pl.ds(i * 16, 16)
            d = dest_v[sl]
            keep = (label_v[sl] <= 0) & (d >= base) & (d < base + HALF)
            idx_v[sl] = jnp.where(keep, d - base, HALF)
        plsc.subcore_barrier()
        s1 = pltpu.async_copy(score_v, bufm_s.at[idx_v], sem)
        s2 = pltpu.async_copy(time_v, buft_s.at[idx_v], sem)
        s1.wait()
        s2.wait()
        plsc.subcore_barrier()
        w1_ = pltpu.async_copy(bufm_s.at[pl.ds(s * STAGE, STAGE)],
                               mem_ref.at[pl.ds(hstage, STAGE)], sem)
        w2_ = pltpu.async_copy(buft_s.at[pl.ds(s * STAGE, STAGE)],
                               tmem_ref.at[pl.ds(hstage, STAGE)], sem)
        w1_.wait()
        w2_.wait()

    return _sc_scatter


def kernel(x1, x2, time, label, w1, b1, w2, b2, memory, time_memory):
    v = w2 @ w1
    va = v[:, :HID]
    vb = v[:, HID:]
    carr = (jnp.dot(b1, w2[0]) + b2[0]).reshape(1, 1)
    label2d = label.astype(jnp.int32).reshape(R, R)
    score, dest2d = _tc_call(x1, x2, va, vb, carr, label2d)
    mem_ref = jax.new_ref(memory)
    tmem_ref = jax.new_ref(time_memory)
    _sc_scatter_fn()(
        dest2d.reshape(B), score.reshape(B), time.reshape(B),
        label.astype(jnp.int32), mem_ref, tmem_ref)
    return score.reshape(B, 1), mem_ref[...], tmem_ref[...]


# speedup vs baseline: 3.0371x; 1.0055x over previous
import functools

import jax
import jax.numpy as jnp
from jax import lax
from jax.experimental import pallas as pl
from jax.experimental.pallas import tpu as pltpu
from jax.experimental.pallas import tpu_sc as plsc

B = 16384
HID = 128
MEM = 1000000
R = 128
NB = 4
BLK = B // NB

_EPS = 1e-12


def _prefix_rows(p):
    for k in (1, 2, 4, 8, 16, 32, 64):
        p = p + jnp.concatenate([jnp.zeros((R, k), jnp.int32), p[:, : R - k]], axis=1)
    return p


def _prefix_col(p):
    for k in (1, 2, 4, 8, 16, 32, 64):
        p = p + jnp.concatenate([jnp.zeros((k, 1), jnp.int32), p[: R - k, :]], axis=0)
    return p


def _tc_body(x1_ref, x2_ref, va_ref, vb_ref, c_ref, label_ref, score_ref, dest_ref):
    x1 = x1_ref[...]
    x2 = x2_ref[...]
    ones_row = jnp.ones((1, HID), jnp.float32)
    dn = (((1,), (1,)), ((), ()))
    d1 = lax.dot_general(va_ref[...], x1, dn, preferred_element_type=jnp.float32)
    s1 = lax.dot_general(ones_row, x1 * x1, dn, preferred_element_type=jnp.float32)
    d2 = lax.dot_general(vb_ref[...], x2, dn, preferred_element_type=jnp.float32)
    s2 = lax.dot_general(ones_row, x2 * x2, dn, preferred_element_type=jnp.float32)
    n1 = jnp.maximum(jnp.sqrt(s1), _EPS)
    n2 = jnp.maximum(jnp.sqrt(s2), _EPS)
    score_ref[...] = d1 / n1 + d2 / n2 + c_ref[0, 0]

    @pl.when(pl.program_id(0) == 0)
    def _():
        m = (label_ref[...] <= 0).astype(jnp.int32)
        pm = _prefix_rows(m)
        rs = pm[:, R - 1 : R]
        ic = _prefix_col(rs)
        off = ic - rs
        dest_ref[...] = jnp.where(m == 1, off + pm - 1, B)


_tc_call = pl.pallas_call(
    _tc_body,
    grid=(NB,),
    in_specs=[
        pl.BlockSpec((BLK, HID), lambda i: (i, 0)),
        pl.BlockSpec((BLK, HID), lambda i: (i, 0)),
        pl.BlockSpec((1, HID), lambda i: (0, 0)),
        pl.BlockSpec((1, HID), lambda i: (0, 0)),
        pl.BlockSpec((1, 1), lambda i: (0, 0)),
        pl.BlockSpec((R, R), lambda i: (0, 0)),
    ],
    out_specs=[
        pl.BlockSpec((1, BLK), lambda i: (0, i)),
        pl.BlockSpec((R, R), lambda i: (0, 0)),
    ],
    out_shape=[
        jax.ShapeDtypeStruct((1, B), jnp.float32),
        jax.ShapeDtypeStruct((R, R), jnp.int32),
    ],
)

_NC = 2
_NS = 16
_NW = _NC * _NS
CHUNK = B // _NS
HALF = B // _NC
STAGE = HALF // _NS


@functools.lru_cache(maxsize=None)
def _sc_scatter_fn():
    mesh = plsc.VectorSubcoreMesh(core_axis_name="c", subcore_axis_name="s")

    @functools.partial(
        pl.kernel,
        mesh=mesh,
        scratch_types=[
            pltpu.VMEM((CHUNK,), jnp.int32),
            pltpu.VMEM((CHUNK,), jnp.float32),
            pltpu.VMEM((CHUNK,), jnp.float32),
            pltpu.VMEM((CHUNK,), jnp.int32),
            pltpu.VMEM_SHARED((HALF + 64,), jnp.float32),
            pltpu.VMEM_SHARED((HALF + 64,), jnp.float32),
            pltpu.SemaphoreType.DMA,
        ],
    )
    def _sc_scatter(dest_hbm, score_hbm, time_hbm, mem_ref, tmem_ref,
                    dest_v, score_v, time_v, idx_v,
                    bufm_s, buft_s, sem):
        c = lax.axis_index("c")
        s = lax.axis_index("s")
        rbase = s * CHUNK
        hstage = c * HALF + s * STAGE
        loads = [
            pltpu.async_copy(dest_hbm.at[pl.ds(rbase, CHUNK)], dest_v, sem),
            pltpu.async_copy(score_hbm.at[pl.ds(rbase, CHUNK)], score_v, sem),
            pltpu.async_copy(time_hbm.at[pl.ds(rbase, CHUNK)], time_v, sem),
            pltpu.async_copy(mem_ref.at[pl.ds(hstage, STAGE)],
                             bufm_s.at[pl.ds(s * STAGE, STAGE)], sem),
            pltpu.async_copy(tmem_ref.at[pl.ds(hstage, STAGE)],
                             buft_s.at[pl.ds(s * STAGE, STAGE)], sem),
        ]
        for ld in loads:
            ld.wait()
        base = c * HALF
        for i in range(CHUNK // 16):
            sl = pl.ds(i * 16, 16)
            d = dest_v[sl]
            keep = (d >= base) & (d < base + HALF)
            idx_v[sl] = jnp.where(keep, d - base, HALF)
        plsc.subcore_barrier()
        s1 = pltpu.async_copy(score_v, bufm_s.at[idx_v], sem)
        s2 = pltpu.async_copy(time_v, buft_s.at[idx_v], sem)
        s1.wait()
        s2.wait()
        plsc.subcore_barrier()
        w1_ = pltpu.async_copy(bufm_s.at[pl.ds(s * STAGE, STAGE)],
                               mem_ref.at[pl.ds(hstage, STAGE)], sem)
        w2_ = pltpu.async_copy(buft_s.at[pl.ds(s * STAGE, STAGE)],
                               tmem_ref.at[pl.ds(hstage, STAGE)], sem)
        w1_.wait()
        w2_.wait()

    return _sc_scatter


def kernel(x1, x2, time, label, w1, b1, w2, b2, memory, time_memory):
    v = w2 @ w1
    va = v[:, :HID]
    vb = v[:, HID:]
    carr = (jnp.dot(b1, w2[0]) + b2[0]).reshape(1, 1)
    label2d = label.astype(jnp.int32).reshape(R, R)
    score, dest2d = _tc_call(x1, x2, va, vb, carr, label2d)
    mem_ref = jax.new_ref(memory)
    tmem_ref = jax.new_ref(time_memory)
    _sc_scatter_fn()(
        dest2d.reshape(B), score.reshape(B), time.reshape(B),
        mem_ref, tmem_ref)
    return score.reshape(B, 1), mem_ref[...], tmem_ref[...]
